# baseline (speedup 1.0000x reference)
import jax
import jax.numpy as jnp
from jax import lax
from jax.experimental import pallas as pl
from jax.experimental.pallas import tpu as pltpu
from jax.experimental.pallas import tpu_sc as plsc

V = 100000
D = 300
B = 256

_LENS = (200, 30, 30, 20, 20, 50, 50)
_NW = 32
_SUB = 80

_mesh = plsc.VectorSubcoreMesh(core_axis_name="c", subcore_axis_name="s")


def _make_main_body(lens):
    chunks = tuple(B * L // _NW for L in lens)
    toff = tuple(sum(chunks[:t]) for t in range(len(lens)))
    total = sum(chunks)
    nt = len(lens)

    def body(*refs):
        idx_hbm = refs[0:nt]
        table_hbm = refs[nt]
        outs = refs[nt + 1:2 * nt + 1]
        idx_v = refs[2 * nt + 1]
        bufs = refs[2 * nt + 2:2 * nt + 4]
        sems = refs[2 * nt + 4:2 * nt + 6]
        sem_idx = refs[2 * nt + 6]

        wid = lax.axis_index("s") * 2 + lax.axis_index("c")

        for t in range(nt):
            pltpu.async_copy(
                idx_hbm[t].at[pl.ds(wid * chunks[t], chunks[t])],
                idx_v.at[pl.ds(toff[t], chunks[t])], sem_idx)
        pltpu.make_async_copy(
            idx_hbm[0].at[pl.ds(0, total)], idx_v, sem_idx).wait()

        def start(t, i, s):
            idx_sl = idx_v.at[pl.ds(toff[t] + i * _SUB, _SUB)]
            pltpu.async_copy(table_hbm.at[idx_sl], bufs[s], sems[s])

        def finish(i, s, out_ref, base):
            pltpu.make_async_copy(
                table_hbm.at[idx_v.at[pl.ds(0, _SUB)]],
                bufs[s], sems[s]).wait()
            pltpu.sync_copy(bufs[s], out_ref.at[pl.ds(base + i * _SUB, _SUB), :])

        for t in range(nt):
            base = wid * chunks[t]
            n_sub = chunks[t] // _SUB
            out_ref = outs[t]

            start(t, 0, 0)

            def sub_step(i, _, t=t, n_sub=n_sub, out_ref=out_ref, base=base):
                @pl.when(i % 2 == 0)
                def _():
                    @pl.when(i + 1 < n_sub)
                    def _():
                        start(t, i + 1, 1)
                    finish(i, 0, out_ref, base)

                @pl.when(i % 2 == 1)
                def _():
                    @pl.when(i + 1 < n_sub)
                    def _():
                        start(t, i + 1, 0)
                    finish(i, 1, out_ref, base)

                return 0

            lax.fori_loop(0, n_sub, sub_step, 0)

    return body, chunks, total


def _make_tail_body():
    chunks = tuple(B * L // _NW for L in _LENS)
    toff = tuple(sum(chunks[:t]) for t in range(7))
    total = sum(chunks)

    def body(*refs):
        idx_hbm = refs[0:7]
        tail_hbm = refs[7]
        outs = refs[8:15]
        idx_v = refs[15]
        bufs = refs[16:18]
        cbufs = refs[18:20]
        sems = refs[20:22]
        sem_idx = refs[22]

        wid = lax.axis_index("s") * 2 + lax.axis_index("c")

        for t in range(7):
            pltpu.async_copy(
                idx_hbm[t].at[pl.ds(wid * chunks[t], chunks[t])],
                idx_v.at[pl.ds(toff[t], chunks[t])], sem_idx)
        pltpu.make_async_copy(
            idx_hbm[0].at[pl.ds(0, total)], idx_v, sem_idx).wait()

        def start(t, i, s):
            idx_sl = idx_v.at[pl.ds(toff[t] + i * _SUB, _SUB)]
            pltpu.async_copy(tail_hbm.at[idx_sl], bufs[s], sems[s])

        def finish(i, s, out_ref, base):
            buf, cbuf = bufs[s], cbufs[s]
            pltpu.make_async_copy(
                tail_hbm.at[idx_v.at[pl.ds(0, _SUB)]], buf, sems[s]).wait()

            def row_step(r, _):
                cbuf[r, pl.ds(0, 16)] = buf[r, pl.ds(0, 16)]
                cbuf[r, pl.ds(16, 16)] = buf[r, pl.ds(16, 16)]
                cbuf[r, pl.ds(28, 16)] = buf[r, pl.ds(28, 16)]
                return 0

            lax.fori_loop(0, _SUB, row_step, 0, unroll=4)
            pltpu.sync_copy(cbuf, out_ref.at[pl.ds(base + i * _SUB, _SUB), :])

        for t in range(7):
            base = wid * chunks[t]
            n_sub = chunks[t] // _SUB
            out_ref = outs[t]

            start(t, 0, 0)

            def sub_step(i, _, t=t, n_sub=n_sub, out_ref=out_ref, base=base):
                @pl.when(i % 2 == 0)
                def _():
                    @pl.when(i + 1 < n_sub)
                    def _():
                        start(t, i + 1, 1)
                    finish(i, 0, out_ref, base)

                @pl.when(i % 2 == 1)
                def _():
                    @pl.when(i + 1 < n_sub)
                    def _():
                        start(t, i + 1, 0)
                    finish(i, 1, out_ref, base)

                return 0

            lax.fori_loop(0, n_sub, sub_step, 0)

    return body, total


def _main_call(table_a, idx_list, lens):
    body, chunks, total = _make_main_body(lens)
    out_type = tuple(
        jax.ShapeDtypeStruct((B * L, 256), jnp.float32) for L in lens
    )
    k = pl.kernel(
        body,
        out_type=out_type,
        mesh=_mesh,
        scratch_types=[pltpu.VMEM((total,), jnp.int32),
                       pltpu.VMEM((_SUB, 256), jnp.float32),
                       pltpu.VMEM((_SUB, 256), jnp.float32)]
        + [pltpu.SemaphoreType.DMA] * 3,
    )
    return k(*idx_list, table_a)


def _tail_call(tail, idx_list):
    body, total = _make_tail_body()
    out_type = tuple(
        jax.ShapeDtypeStruct((B * L, 44), jnp.float32) for L in _LENS
    )
    k = pl.kernel(
        body,
        out_type=out_type,
        mesh=_mesh,
        scratch_types=[pltpu.VMEM((total,), jnp.int32),
                       pltpu.VMEM((_SUB, 128), jnp.float32),
                       pltpu.VMEM((_SUB, 128), jnp.float32),
                       pltpu.VMEM((_SUB, 44), jnp.float32),
                       pltpu.VMEM((_SUB, 44), jnp.float32)]
        + [pltpu.SemaphoreType.DMA] * 3,
    )
    return k(*idx_list, tail)


@jax.jit
def _gather_all(table, *idx_flat):
    tail = jnp.pad(table[:, 256:300], ((0, 0), (0, 84)))
    table_a = table[:, :256]

    tails = _tail_call(tail, idx_flat)
    mains_b = _main_call(table_a, idx_flat[0:1], _LENS[0:1])
    mains_c = _main_call(table_a, idx_flat[5:7], _LENS[5:7])
    mains_d = _main_call(table_a, idx_flat[1:5], _LENS[1:5])

    mains = (mains_b[0], mains_d[0], mains_d[1], mains_d[2], mains_d[3],
             mains_c[0], mains_c[1])
    return tuple(
        jnp.concatenate([m, t], axis=1) for m, t in zip(mains, tails)
    )


def kernel(passage, passage_mask, question, question_mask, questioninfo,
           questioninfo_mask, answer1, answer1_mask, answer2, answer2_mask,
           qanswer1, qanswer1_mask, qanswer2, qanswer2_mask, table):
    idxs = (passage, question, questioninfo, answer1, answer2, qanswer1,
            qanswer2)
    flat = tuple(a.reshape(-1) for a in idxs)
    embs = _gather_all(table, *flat)
    embs = tuple(e.reshape(a.shape[0], a.shape[1], D)
                 for e, a in zip(embs, idxs))
    return (embs[0], passage_mask, embs[1], question_mask, embs[2],
            questioninfo_mask, embs[3], answer1_mask, embs[4], answer2_mask,
            embs[5], qanswer1_mask, embs[6], qanswer2_mask)
